# Initial kernel scaffold; baseline (speedup 1.0000x reference)
import jax
import jax.numpy as jnp
from jax.experimental import pallas as pl


def kernel(t, out, targets, stimulation_strength):
    raise NotImplementedError("write your pallas kernel here")


# speedup vs baseline: 1.3378x; 1.3378x over previous
import dataclasses
import functools

import jax
import jax.numpy as jnp
from jax import lax
from jax.experimental import pallas as pl
from jax.experimental.pallas import tpu as pltpu
from jax.experimental.pallas import tpu_sc as plsc

_RATE = 0.1

_L = 16
_NC = 2
_NS = 16
_NW = _NC * _NS
_N = 1_000_000
_ROWS = _N // _L
_RPW = (_ROWS // _NW) // 8 * 8
_MAIN = _RPW * _NW
_REM = _ROWS - _MAIN
_NT = 128

def _apply_adds(buf, tgt_v, stim_v, tmod_v, base, nrows):
    gate_zero = tmod_v[...] == 0.0
    for j in range(_NT // _L):
        t = tgt_v[pl.ds(j * _L, _L)]
        s = stim_v[pl.ds(j * _L, _L)]
        s = jnp.where(gate_zero, s, jnp.zeros_like(s))
        row = lax.div(t, _L) - base
        col = lax.rem(t, _L)
        inb = (row >= 0) & (row < nrows)
        row_c = jnp.minimum(jnp.maximum(row, 0), nrows - 1)
        plsc.addupdate_scatter(buf, [row_c, col], s, mask=inb)


@functools.lru_cache(maxsize=1)
def _build_stim_kernel():
    mesh = plsc.VectorSubcoreMesh(
        core_axis_name="c", subcore_axis_name="s",
        num_cores=_NC, num_subcores=_NS,
    )
    cp = pltpu.CompilerParams()
    if "needs_layout_passes" in pltpu.CompilerParams.__dataclass_fields__:
        cp = dataclasses.replace(cp, needs_layout_passes=False)
    if "use_tc_tiling_on_sc" in pltpu.CompilerParams.__dataclass_fields__:
        cp = dataclasses.replace(cp, use_tc_tiling_on_sc=False)

    @functools.partial(
        pl.kernel,
        compiler_params=cp,
        out_type=jax.ShapeDtypeStruct((_ROWS, _L), jnp.float32),
        mesh=mesh,
        scratch_types=[
            pltpu.VMEM((_RPW, _L), jnp.float32),
            pltpu.VMEM((_REM, _L), jnp.float32),
            pltpu.VMEM((_NT,), jnp.int32),
            pltpu.VMEM((_NT,), jnp.float32),
            pltpu.VMEM((_L,), jnp.float32),
            pltpu.SemaphoreType.DMA,
        ],
    )
    def _stim_kernel(x_hbm, tgt_hbm, stim_hbm, tmod_hbm, o_hbm,
                     buf, rbuf, tgt_v, stim_v, tmod_v, sem):
        wid = lax.axis_index("s") * _NC + lax.axis_index("c")
        base = wid * _RPW
        cin = pltpu.async_copy(x_hbm.at[pl.ds(base, _RPW)], buf, sem)
        pltpu.sync_copy(tgt_hbm, tgt_v)
        pltpu.sync_copy(stim_hbm, stim_v)
        pltpu.sync_copy(tmod_hbm, tmod_v)
        cin.wait()
        _apply_adds(buf, tgt_v, stim_v, tmod_v, base, _RPW)
        pltpu.async_copy(buf, o_hbm.at[pl.ds(base, _RPW)], sem).wait()

        @pl.when(wid == _NW - 1)
        def _tail():
            pltpu.sync_copy(x_hbm.at[pl.ds(_MAIN, _REM)], rbuf)
            _apply_adds(rbuf, tgt_v, stim_v, tmod_v, _MAIN, _REM)
            pltpu.sync_copy(rbuf, o_hbm.at[pl.ds(_MAIN, _REM)])

    return _stim_kernel


def kernel(t, out, targets, stimulation_strength):
    tmod = (t % (1.0 / _RATE)).astype(jnp.float32)
    tmod_vec = jnp.broadcast_to(tmod, (_L,))
    x2d = out.reshape(_ROWS, _L)
    tgt = targets.astype(jnp.int32)
    o2d = _build_stim_kernel()(x2d, tgt, stimulation_strength, tmod_vec)
    return o2d.reshape(_N)
